# initial kernel scaffold (unmeasured)
import jax
import jax.numpy as jnp
from jax import lax
from jax.experimental import pallas as pl
from jax.experimental.pallas import tpu as pltpu


def kernel(
    x,
):
    def body(*refs):
        pass

    out_shape = jax.ShapeDtypeStruct(..., jnp.float32)
    return pl.pallas_call(body, out_shape=out_shape)(...)


# baseline (device time: 19214 ns/iter reference)
import jax
import jax.numpy as jnp
from jax import lax
from jax.experimental import pallas as pl
from jax.experimental.pallas import tpu as pltpu

N_Z = 4


def kernel(x):
    m_per, n = x.shape

    def body(x_ref, out_ref, send_sems, recv_sems):
        my_x = lax.axis_index("x")
        my_y = lax.axis_index("y")
        my_z = lax.axis_index("z")
        left = (my_z - 1) % N_Z
        right = (my_z + 1) % N_Z

        barrier_sem = pltpu.get_barrier_semaphore()
        for nbr in (left, right):
            pl.semaphore_signal(
                barrier_sem,
                inc=1,
                device_id=(my_x, my_y, nbr),
                device_id_type=pl.DeviceIdType.MESH,
            )
        pl.semaphore_wait(barrier_sem, 2)

        out_ref[pl.ds(my_z * m_per, m_per), :] = x_ref[...]

        for h in range(N_Z - 1):
            src_chunk = (my_z - h) % N_Z
            rdma = pltpu.make_async_remote_copy(
                src_ref=out_ref.at[pl.ds(src_chunk * m_per, m_per), :],
                dst_ref=out_ref.at[pl.ds(src_chunk * m_per, m_per), :],
                send_sem=send_sems.at[h],
                recv_sem=recv_sems.at[h],
                device_id=(my_x, my_y, right),
                device_id_type=pl.DeviceIdType.MESH,
            )
            rdma.start()
            rdma.wait()

    return pl.pallas_call(
        body,
        out_shape=jax.ShapeDtypeStruct((N_Z * m_per, n), x.dtype),
        in_specs=[pl.BlockSpec(memory_space=pltpu.VMEM)],
        out_specs=pl.BlockSpec(memory_space=pltpu.VMEM),
        scratch_shapes=[
            pltpu.SemaphoreType.DMA((N_Z - 1,)),
            pltpu.SemaphoreType.DMA((N_Z - 1,)),
        ],
        compiler_params=pltpu.CompilerParams(collective_id=0),
    )(x)


# device time: 18346 ns/iter; 1.0473x vs baseline; 1.0473x over previous
import jax
import jax.numpy as jnp
from jax import lax
from jax.experimental import pallas as pl
from jax.experimental.pallas import tpu as pltpu

N_Z = 4


def kernel(x):
    m_per, n = x.shape

    def body(x_ref, out_ref, sd_sems, rd_sems, su_sems, ru_sems):
        my_x = lax.axis_index("x")
        my_y = lax.axis_index("y")
        my_z = lax.axis_index("z")
        dn = jnp.maximum(my_z - 1, 0)
        up = jnp.minimum(my_z + 1, N_Z - 1)

        def cs(c):
            return pl.ds(c * m_per, m_per)

        def clamp(c):
            return jnp.clip(c, 0, N_Z - 1)

        def dn_send(k):
            c = clamp(my_z + k)
            return pltpu.make_async_remote_copy(
                src_ref=out_ref.at[cs(c), :],
                dst_ref=out_ref.at[cs(c), :],
                send_sem=sd_sems.at[k],
                recv_sem=rd_sems.at[k],
                device_id=(my_x, my_y, dn),
                device_id_type=pl.DeviceIdType.MESH,
            )

        def dn_recv(k):
            c = clamp(my_z + 1 + k)
            return pltpu.make_async_remote_copy(
                src_ref=out_ref.at[cs(c), :],
                dst_ref=out_ref.at[cs(c), :],
                send_sem=sd_sems.at[k],
                recv_sem=rd_sems.at[k],
                device_id=(my_x, my_y, up),
                device_id_type=pl.DeviceIdType.MESH,
            )

        def up_send(k):
            c = clamp(my_z - k)
            return pltpu.make_async_remote_copy(
                src_ref=out_ref.at[cs(c), :],
                dst_ref=out_ref.at[cs(c), :],
                send_sem=su_sems.at[k],
                recv_sem=ru_sems.at[k],
                device_id=(my_x, my_y, up),
                device_id_type=pl.DeviceIdType.MESH,
            )

        def up_recv(k):
            c = clamp(my_z - 1 - k)
            return pltpu.make_async_remote_copy(
                src_ref=out_ref.at[cs(c), :],
                dst_ref=out_ref.at[cs(c), :],
                send_sem=su_sems.at[k],
                recv_sem=ru_sems.at[k],
                device_id=(my_x, my_y, dn),
                device_id_type=pl.DeviceIdType.MESH,
            )

        def dn_send_valid(k):
            return (my_z >= 1) & (my_z + k <= N_Z - 1)

        def up_send_valid(k):
            return (my_z <= N_Z - 2) & (my_z - k >= 0)

        def dn_recv_valid(k):
            return my_z <= N_Z - 2 - k

        def up_recv_valid(k):
            return my_z >= 1 + k

        barrier_sem = pltpu.get_barrier_semaphore()

        @pl.when(my_z >= 1)
        def _():
            pl.semaphore_signal(
                barrier_sem, inc=1,
                device_id=(my_x, my_y, dn),
                device_id_type=pl.DeviceIdType.MESH,
            )

        @pl.when(my_z <= N_Z - 2)
        def _():
            pl.semaphore_signal(
                barrier_sem, inc=1,
                device_id=(my_x, my_y, up),
                device_id_type=pl.DeviceIdType.MESH,
            )

        @pl.when(my_z >= 1)
        def _():
            pl.semaphore_wait(barrier_sem, 1)

        @pl.when(my_z <= N_Z - 2)
        def _():
            pl.semaphore_wait(barrier_sem, 1)

        out_ref[cs(my_z), :] = x_ref[...]

        for k in range(N_Z - 1):
            if k > 0:
                @pl.when(dn_recv_valid(k - 1))
                def _(k=k):
                    dn_recv(k - 1).wait_recv()

            @pl.when(dn_send_valid(k))
            def _(k=k):
                dn_send(k).start()

            if k > 0:
                @pl.when(up_recv_valid(k - 1))
                def _(k=k):
                    up_recv(k - 1).wait_recv()

            @pl.when(up_send_valid(k))
            def _(k=k):
                up_send(k).start()

        @pl.when(dn_recv_valid(N_Z - 2))
        def _():
            dn_recv(N_Z - 2).wait_recv()

        @pl.when(up_recv_valid(N_Z - 2))
        def _():
            up_recv(N_Z - 2).wait_recv()

        for k in range(N_Z - 1):
            @pl.when(dn_send_valid(k))
            def _(k=k):
                dn_send(k).wait_send()

            @pl.when(up_send_valid(k))
            def _(k=k):
                up_send(k).wait_send()

    return pl.pallas_call(
        body,
        out_shape=jax.ShapeDtypeStruct((N_Z * m_per, n), x.dtype),
        in_specs=[pl.BlockSpec(memory_space=pltpu.VMEM)],
        out_specs=pl.BlockSpec(memory_space=pltpu.VMEM),
        scratch_shapes=[
            pltpu.SemaphoreType.DMA((N_Z - 1,)),
            pltpu.SemaphoreType.DMA((N_Z - 1,)),
            pltpu.SemaphoreType.DMA((N_Z - 1,)),
            pltpu.SemaphoreType.DMA((N_Z - 1,)),
        ],
        compiler_params=pltpu.CompilerParams(collective_id=0),
    )(x)


# device time: 15924 ns/iter; 1.2066x vs baseline; 1.1521x over previous
import jax
import jax.numpy as jnp
from jax import lax
from jax.experimental import pallas as pl
from jax.experimental.pallas import tpu as pltpu

N_Z = 4


def kernel(x):
    m_per, n = x.shape

    def body(x_ref, out_ref, send_sems, recv_sems):
        my_x = lax.axis_index("x")
        my_y = lax.axis_index("y")
        my_z = lax.axis_index("z")

        def cs(c):
            return pl.ds(c * m_per, m_per)

        barrier_sem = pltpu.get_barrier_semaphore()
        for t in range(N_Z):
            @pl.when(my_z != t)
            def _(t=t):
                pl.semaphore_signal(
                    barrier_sem, inc=1,
                    device_id=(my_x, my_y, t),
                    device_id_type=pl.DeviceIdType.MESH,
                )
        pl.semaphore_wait(barrier_sem, N_Z - 1)

        out_ref[cs(my_z), :] = x_ref[...]

        def send_to(t):
            return pltpu.make_async_remote_copy(
                src_ref=out_ref.at[cs(my_z), :],
                dst_ref=out_ref.at[cs(my_z), :],
                send_sem=send_sems.at[t],
                recv_sem=recv_sems.at[my_z],
                device_id=(my_x, my_y, t),
                device_id_type=pl.DeviceIdType.MESH,
            )

        def recv_from(o):
            return pltpu.make_async_remote_copy(
                src_ref=out_ref.at[cs(o), :],
                dst_ref=out_ref.at[cs(o), :],
                send_sem=send_sems.at[o],
                recv_sem=recv_sems.at[o],
                device_id=(my_x, my_y, o),
                device_id_type=pl.DeviceIdType.MESH,
            )

        for t in range(N_Z):
            @pl.when(my_z != t)
            def _(t=t):
                send_to(t).start()

        for o in range(N_Z):
            @pl.when(my_z != o)
            def _(o=o):
                recv_from(o).wait_recv()

        for t in range(N_Z):
            @pl.when(my_z != t)
            def _(t=t):
                send_to(t).wait_send()

    return pl.pallas_call(
        body,
        out_shape=jax.ShapeDtypeStruct((N_Z * m_per, n), x.dtype),
        in_specs=[pl.BlockSpec(memory_space=pltpu.VMEM)],
        out_specs=pl.BlockSpec(memory_space=pltpu.VMEM),
        scratch_shapes=[
            pltpu.SemaphoreType.DMA((N_Z,)),
            pltpu.SemaphoreType.DMA((N_Z,)),
        ],
        compiler_params=pltpu.CompilerParams(collective_id=0),
    )(x)


# device time: 14748 ns/iter; 1.3028x vs baseline; 1.0797x over previous
import jax
import jax.numpy as jnp
from jax import lax
from jax.experimental import pallas as pl
from jax.experimental.pallas import tpu as pltpu

N_Z = 4


def kernel(x):
    m_per, n = x.shape

    def body(x_ref, out_ref, send_sems, recv_sems):
        my_x = lax.axis_index("x")
        my_y = lax.axis_index("y")
        my_z = lax.axis_index("z")

        def cs(c):
            return pl.ds(c * m_per, m_per)

        barrier_sem = pltpu.get_barrier_semaphore()
        pl.semaphore_signal(barrier_sem, inc=1)
        pl.semaphore_wait(barrier_sem, 1)

        out_ref[cs(my_z), :] = x_ref[...]

        def send_to(t):
            return pltpu.make_async_remote_copy(
                src_ref=out_ref.at[cs(my_z), :],
                dst_ref=out_ref.at[cs(my_z), :],
                send_sem=send_sems.at[t],
                recv_sem=recv_sems.at[my_z],
                device_id=(my_x, my_y, t),
                device_id_type=pl.DeviceIdType.MESH,
            )

        def recv_from(o):
            return pltpu.make_async_remote_copy(
                src_ref=out_ref.at[cs(o), :],
                dst_ref=out_ref.at[cs(o), :],
                send_sem=send_sems.at[o],
                recv_sem=recv_sems.at[o],
                device_id=(my_x, my_y, o),
                device_id_type=pl.DeviceIdType.MESH,
            )

        for t in range(N_Z):
            @pl.when(my_z != t)
            def _(t=t):
                send_to(t).start()

        for o in range(N_Z):
            @pl.when(my_z != o)
            def _(o=o):
                recv_from(o).wait_recv()

        for t in range(N_Z):
            @pl.when(my_z != t)
            def _(t=t):
                send_to(t).wait_send()

    return pl.pallas_call(
        body,
        out_shape=jax.ShapeDtypeStruct((N_Z * m_per, n), x.dtype),
        in_specs=[pl.BlockSpec(memory_space=pltpu.VMEM)],
        out_specs=pl.BlockSpec(memory_space=pltpu.VMEM),
        scratch_shapes=[
            pltpu.SemaphoreType.DMA((N_Z,)),
            pltpu.SemaphoreType.DMA((N_Z,)),
        ],
        compiler_params=pltpu.CompilerParams(collective_id=0),
    )(x)
